# baseline (speedup 1.0000x reference)
import dataclasses
import functools

import jax
import jax.numpy as jnp
from jax import lax
from jax.experimental import pallas as pl
from jax.experimental.pallas import tpu as pltpu
from jax.experimental.pallas import tpu_sc as plsc

B = 16384
L = 50
DIM = 32
NC = 2
NS = 16
NW = NC * NS
NBT = B // 128
BT_PER_W = NBT // NW


def kernel(input, emb_weight):
    idx2d = input.T.astype(jnp.int32)

    @functools.partial(
        pl.kernel,
        out_type=jax.ShapeDtypeStruct((L, 4, 128, 8, 128), jnp.float32),
        mesh=plsc.VectorSubcoreMesh(core_axis_name="c", subcore_axis_name="s"),
        compiler_params=dataclasses.replace(
            pltpu.CompilerParams(use_tc_tiling_on_sc=False),
            needs_layout_passes=False,
        ),
        scratch_types=[
            pltpu.VMEM((L, 128), jnp.int32),
            pltpu.VMEM((128, DIM), jnp.float32),
            pltpu.VMEM((128, DIM), jnp.float32),
            pltpu.VMEM((4, 8, 128), jnp.float32),
            pltpu.VMEM((4, 8, 128), jnp.float32),
            pltpu.SemaphoreType.DMA,
            pltpu.SemaphoreType.DMA,
            pltpu.SemaphoreType.DMA,
            pltpu.SemaphoreType.DMA,
        ],
    )
    def gather_kernel(table_hbm, idx_hbm, out_hbm,
                      idxv, r0, r1, t0, t1, g0, g1, w0, w1):
        rows = [r0, r1]
        ts = [t0, t1]
        gs = [g0, g1]
        ws = [w0, w1]

        wid = lax.axis_index("s") * NC + lax.axis_index("c")
        iota = lax.iota(jnp.int32, 16)

        for k_bt in range(BT_PER_W):
            bt = wid * BT_PER_W + k_bt
            pltpu.sync_copy(idx_hbm.at[:, pl.ds(bt * 128, 128)], idxv)
            pltpu.async_copy(table_hbm.at[idxv.at[0]], rows[0], gs[0])

            def step(l, par):
                pltpu.make_async_copy(
                    table_hbm.at[idxv.at[0]], rows[par], gs[par]
                ).wait()

                @pl.when(l < L - 1)
                def _():
                    pltpu.async_copy(
                        table_hbm.at[idxv.at[l + 1]], rows[1 - par], gs[1 - par]
                    )

                @pl.when(l >= 2)
                def _():
                    pltpu.make_async_copy(
                        ts[par], out_hbm.at[0, :, 0], ws[par]
                    ).wait()

                @pl.loop(0, DIM)
                def _(c):
                    ct = c // 8
                    cs = c % 8
                    cvec = jnp.full((16,), c, jnp.int32)
                    for bl0 in range(0, 128, 16):
                        v = plsc.load_gather(rows[par], [iota + bl0, cvec])
                        ts[par][ct, cs, pl.ds(bl0, 16)] = v

                pltpu.async_copy(ts[par], out_hbm.at[l, :, bt], ws[par])

            @pl.loop(0, L, step=2)
            def _(l0):
                step(l0, 0)
                step(l0 + 1, 1)

            pltpu.make_async_copy(ts[0], out_hbm.at[0, :, 0], ws[0]).wait()
            pltpu.make_async_copy(ts[1], out_hbm.at[0, :, 0], ws[1]).wait()

    out5d = gather_kernel(emb_weight, idx2d)
    return out5d.transpose(2, 4, 0, 1, 3).reshape(B, L, DIM)
